# Initial kernel scaffold; baseline (speedup 1.0000x reference)
import jax
import jax.numpy as jnp
from jax.experimental import pallas as pl


def kernel(data_x, data_edge_index, params):
    raise NotImplementedError("write your pallas kernel here")


# speedup vs baseline: 4.4044x; 4.4044x over previous
import functools
import math

import jax
import jax.numpy as jnp
from jax import lax
from jax.experimental import pallas as pl
from jax.experimental.pallas import tpu as pltpu
from jax.experimental.pallas import tpu_sc as plsc

_NC = 2
_NS = 16
_NW = _NC * _NS
_CH = 128


@functools.lru_cache(maxsize=None)
def _make_segsum(n_rows, n_chunks, width):
    zrows = ((n_rows + 1 + _NS * 8 - 1) // (_NS * 8)) * 8
    n_acc = zrows * _NS
    mesh = plsc.VectorSubcoreMesh(
        core_axis_name="c", subcore_axis_name="s",
        num_cores=_NC, num_subcores=_NS)

    @functools.partial(
        pl.kernel,
        out_type=jax.ShapeDtypeStruct((_NC, n_acc, width), jnp.float32),
        mesh=mesh,
        scratch_types=[
            pltpu.VMEM((_CH,), jnp.int32),
            pltpu.VMEM((_CH,), jnp.int32),
            pltpu.VMEM((_CH, width), jnp.float32),
            pltpu.VMEM((zrows, width), jnp.float32),
            pltpu.VMEM_SHARED((n_acc, width), jnp.float32),
            pltpu.SemaphoreType.DMA,
        ],
        compiler_params=pltpu.CompilerParams(use_tc_tiling_on_sc=False),
    )
    def segsum(p_hbm, src_hbm, dst_hbm, out_hbm, sidx, didx, rows, zbuf,
               acc, sem):
        cid = lax.axis_index("c")
        sid = lax.axis_index("s")
        wid = sid * _NC + cid
        base = wid * n_chunks * _CH

        zv = jnp.zeros((16,), jnp.float32)
        nsub = width // 16
        def zero_row(i, carry):
            zbuf[i // nsub, pl.ds((i % nsub) * 16, 16)] = zv
            return carry
        lax.fori_loop(0, zrows * nsub, zero_row, 0)
        pltpu.sync_copy(zbuf, acc.at[pl.ds(sid * zrows, zrows)])
        plsc.subcore_barrier()

        def chunk(c, carry):
            pltpu.sync_copy(src_hbm.at[pl.ds(base + c * _CH, _CH)], sidx)
            pltpu.sync_copy(dst_hbm.at[pl.ds(base + c * _CH, _CH)], didx)
            pltpu.async_copy(p_hbm.at[sidx], rows, sem).wait()
            pltpu.sync_copy(rows, acc.at[didx], add=True)
            return carry
        lax.fori_loop(0, n_chunks, chunk, 0)
        plsc.subcore_barrier()

        pltpu.sync_copy(acc.at[pl.ds(sid * zrows, zrows)],
                        out_hbm.at[cid, pl.ds(sid * zrows, zrows)])

    return segsum


def _dot1x(a, w):
    return jnp.dot(a.astype(jnp.bfloat16), w.astype(jnp.bfloat16),
                   preferred_element_type=jnp.float32)


def _bn_relu(h, g, b):
    mean = jnp.mean(h, axis=0, keepdims=True)
    var = jnp.mean((h - mean) ** 2, axis=0, keepdims=True)
    return jnp.maximum((h - mean) / jnp.sqrt(var + 1e-5) * g + b, 0.0)


def _mlp_core(x_ref, s_ref, w1, b1, g1, be1, w2, b2, g2, be2, w3, b3):
    hin = x_ref[...] + s_ref[0] + s_ref[1]
    h = _dot1x(hin, w1[...]) + b1[...]
    h = _bn_relu(h, g1[...], be1[...])
    h = _dot1x(h, w2[...]) + b2[...]
    h = _bn_relu(h, g2[...], be2[...])
    return _dot1x(h, w3[...]) + b3[...]


def _mlp_relu_body(x_ref, s_ref, w1, b1, g1, be1, w2, b2, g2, be2, w3, b3,
                   o_ref):
    o_ref[...] = jnp.maximum(
        _mlp_core(x_ref, s_ref, w1, b1, g1, be1, w2, b2, g2, be2, w3, b3),
        0.0)


def _mlp_last_body(x_ref, s_ref, w1, b1, g1, be1, w2, b2, g2, be2, w3, b3,
                   rw, rb, o_ref):
    h = _mlp_core(x_ref, s_ref, w1, b1, g1, be1, w2, b2, g2, be2, w3, b3)
    h = h.astype(jnp.bfloat16).astype(jnp.float32)
    o_ref[...] = jnp.dot(h, rw[...], preferred_element_type=jnp.float32,
                         precision=lax.Precision.HIGHEST) + rb[...]


def _tc_call(body, out_shape, *args):
    return pl.pallas_call(
        body, out_shape=jax.ShapeDtypeStruct(out_shape, jnp.float32),
    )(*args)


def _row(v):
    return v.reshape(1, -1)


def kernel(data_x, data_edge_index, params):
    n, d_in = data_x.shape
    h = params['conv1']['W1'].shape[1]
    e = data_edge_index.shape[1]

    n_chunks = math.ceil(e / (_NW * _CH))
    e_pad = _NW * n_chunks * _CH
    src = data_edge_index[0]
    dst = data_edge_index[1]
    if e_pad != e:
        pad = e_pad - e
        src = jnp.concatenate([src, jnp.zeros((pad,), jnp.int32)])
        dst = jnp.concatenate([dst, jnp.full((pad,), n, jnp.int32)])

    n_acc = _NS * (((n + 1 + _NS * 8 - 1) // (_NS * 8)) * 8)

    def seg(p):
        w = p.shape[1]
        outs = []
        for c0 in range(0, w, 64):
            blk = p[:, c0:c0 + 64]
            wb = blk.shape[1]
            blk = jnp.pad(blk, ((0, n_acc - n), (0, 0)))
            outs.append(_make_segsum(n, n_chunks, wb)(blk, src, dst)[:, :n])
        return outs[0] if len(outs) == 1 else jnp.concatenate(outs, axis=2)

    def mlp_args(c):
        p = params[c]
        return (p['W1'], _row(p['b1']), _row(p['g1']), _row(p['be1']),
                p['W2'], _row(p['b2']), _row(p['g2']), _row(p['be2']),
                p['W3'], _row(p['b3']))

    x1 = _tc_call(_mlp_relu_body, (n, h), data_x, seg(data_x),
                  *mlp_args('conv1'))
    x2 = _tc_call(_mlp_relu_body, (n, h), x1, seg(x1), *mlp_args('conv2'))
    out = _tc_call(_mlp_last_body, (n, 1), x2, seg(x2), *mlp_args('conv3'),
                   params['rates_W'], _row(params['rates_b']))
    return out


# speedup vs baseline: 6.1857x; 1.4045x over previous
import functools
import math

import jax
import jax.numpy as jnp
from jax import lax
from jax.experimental import pallas as pl
from jax.experimental.pallas import tpu as pltpu
from jax.experimental.pallas import tpu_sc as plsc

_NC = 2
_NS = 16
_NW = _NC * _NS
_CH = 128


@functools.lru_cache(maxsize=None)
def _make_segsum(n_rows, n_chunks, width):
    zrows = ((n_rows + 1 + _NS * 8 - 1) // (_NS * 8)) * 8
    n_acc = zrows * _NS
    mesh = plsc.VectorSubcoreMesh(
        core_axis_name="c", subcore_axis_name="s",
        num_cores=_NC, num_subcores=_NS)

    @functools.partial(
        pl.kernel,
        out_type=jax.ShapeDtypeStruct((_NC, n_acc, width), jnp.float32),
        mesh=mesh,
        scratch_types=[
            pltpu.VMEM((2, _CH), jnp.int32),
            pltpu.VMEM((2, _CH), jnp.int32),
            pltpu.VMEM((2, _CH, width), jnp.float32),
            pltpu.VMEM((zrows, width), jnp.float32),
            pltpu.VMEM_SHARED((n_acc, width), jnp.float32),
            pltpu.SemaphoreType.DMA,
        ],
        compiler_params=pltpu.CompilerParams(use_tc_tiling_on_sc=False),
    )
    def segsum(p_hbm, src_hbm, dst_hbm, out_hbm, sidx, didx, rows, zbuf,
               acc, sem):
        cid = lax.axis_index("c")
        sid = lax.axis_index("s")
        wid = sid * _NC + cid
        base = wid * n_chunks * _CH

        zv = jnp.zeros((16,), jnp.float32)
        nsub = width // 16
        def zero_row(i, carry):
            zbuf[i // nsub, pl.ds((i % nsub) * 16, 16)] = zv
            return carry
        lax.fori_loop(0, zrows * nsub, zero_row, 0)
        pltpu.sync_copy(zbuf, acc.at[pl.ds(sid * zrows, zrows)])
        plsc.subcore_barrier()

        def fetch(c):
            b = c % 2
            pltpu.sync_copy(src_hbm.at[pl.ds(base + c * _CH, _CH)],
                            sidx.at[b])
            pltpu.sync_copy(dst_hbm.at[pl.ds(base + c * _CH, _CH)],
                            didx.at[b])
            pltpu.async_copy(p_hbm.at[sidx.at[b]], rows.at[b], sem)

        fetch(0)
        def chunk(c, carry):
            b = c % 2
            @pl.when(c + 1 < n_chunks)
            def _():
                fetch(c + 1)
            pltpu.make_async_copy(p_hbm.at[sidx.at[b]], rows.at[b],
                                  sem).wait()
            pltpu.sync_copy(rows.at[b], acc.at[didx.at[b]], add=True)
            return carry
        lax.fori_loop(0, n_chunks, chunk, 0)
        plsc.subcore_barrier()

        pltpu.sync_copy(acc.at[pl.ds(sid * zrows, zrows)],
                        out_hbm.at[cid, pl.ds(sid * zrows, zrows)])

    return segsum


def _dot1x(a, w):
    return jnp.dot(a.astype(jnp.bfloat16), w.astype(jnp.bfloat16),
                   preferred_element_type=jnp.float32)


def _bn_relu(h, g, b):
    mean = jnp.mean(h, axis=0, keepdims=True)
    var = jnp.mean((h - mean) ** 2, axis=0, keepdims=True)
    return jnp.maximum((h - mean) / jnp.sqrt(var + 1e-5) * g + b, 0.0)


def _mlp_core(x_ref, s_ref, w1, b1, g1, be1, w2, b2, g2, be2, w3, b3):
    hin = x_ref[...] + s_ref[0] + s_ref[1]
    h = _dot1x(hin, w1[...]) + b1[...]
    h = _bn_relu(h, g1[...], be1[...])
    h = _dot1x(h, w2[...]) + b2[...]
    h = _bn_relu(h, g2[...], be2[...])
    return _dot1x(h, w3[...]) + b3[...]


def _mlp_relu_body(x_ref, s_ref, w1, b1, g1, be1, w2, b2, g2, be2, w3, b3,
                   o_ref):
    o_ref[...] = jnp.maximum(
        _mlp_core(x_ref, s_ref, w1, b1, g1, be1, w2, b2, g2, be2, w3, b3),
        0.0)


def _mlp_last_body(x_ref, s_ref, w1, b1, g1, be1, w2, b2, g2, be2, w3, b3,
                   rw, rb, o_ref):
    h = _mlp_core(x_ref, s_ref, w1, b1, g1, be1, w2, b2, g2, be2, w3, b3)
    h = h.astype(jnp.bfloat16).astype(jnp.float32)
    o_ref[...] = jnp.dot(h, rw[...], preferred_element_type=jnp.float32,
                         precision=lax.Precision.HIGHEST) + rb[...]


def _tc_call(body, out_shape, *args):
    return pl.pallas_call(
        body, out_shape=jax.ShapeDtypeStruct(out_shape, jnp.float32),
    )(*args)


def _row(v):
    return v.reshape(1, -1)


def kernel(data_x, data_edge_index, params):
    n, d_in = data_x.shape
    h = params['conv1']['W1'].shape[1]
    e = data_edge_index.shape[1]

    n_chunks = math.ceil(e / (_NW * _CH))
    e_pad = _NW * n_chunks * _CH
    src = data_edge_index[0]
    dst = data_edge_index[1]
    if e_pad != e:
        pad = e_pad - e
        src = jnp.concatenate([src, jnp.zeros((pad,), jnp.int32)])
        dst = jnp.concatenate([dst, jnp.full((pad,), n, jnp.int32)])

    n_acc = _NS * (((n + 1 + _NS * 8 - 1) // (_NS * 8)) * 8)

    def seg(p):
        w = p.shape[1]
        outs = []
        for c0 in range(0, w, 64):
            blk = p[:, c0:c0 + 64]
            wb = blk.shape[1]
            blk = jnp.pad(blk, ((0, n_acc - n), (0, 0)))
            outs.append(_make_segsum(n, n_chunks, wb)(blk, src, dst)[:, :n])
        return outs[0] if len(outs) == 1 else jnp.concatenate(outs, axis=2)

    def mlp_args(c):
        p = params[c]
        return (p['W1'], _row(p['b1']), _row(p['g1']), _row(p['be1']),
                p['W2'], _row(p['b2']), _row(p['g2']), _row(p['be2']),
                p['W3'], _row(p['b3']))

    x1 = _tc_call(_mlp_relu_body, (n, h), data_x, seg(data_x),
                  *mlp_args('conv1'))
    x2 = _tc_call(_mlp_relu_body, (n, h), x1, seg(x1), *mlp_args('conv2'))
    out = _tc_call(_mlp_last_body, (n, 1), x2, seg(x2), *mlp_args('conv3'),
                   params['rates_W'], _row(params['rates_b']))
    return out
